# baseline (speedup 1.0000x reference)
import functools

import jax
import jax.numpy as jnp
from jax import lax
from jax.experimental import pallas as pl
from jax.experimental.pallas import tpu as pltpu
from jax.experimental.pallas import tpu_sc as plsc

_NUM_EMB = 128
_DIM = 64
_BATCH = 64
_SEQ = 1024
_P = 8
_NSTEPS = _BATCH // _P
_N_ROWS = _BATCH * _SEQ
_INV_ND = 1.0 / (_N_ROWS * _DIM)


def _vq_body(x_ref, cb_ref, idx_ref, loss_ref, perp_ref,
             counts_acc, loss_acc):
    i = pl.program_id(0)
    cb = cb_ref[...]
    c2 = jnp.sum(cb * cb, axis=1, keepdims=True)
    rev = 128.0 - jax.lax.broadcasted_iota(
        jnp.int32, (_NUM_EMB, 1), 0).astype(jnp.float32)
    ones_s = jnp.ones((_SEQ, 1), jnp.float32)

    @pl.when(i == 0)
    def _init():
        counts_acc[...] = jnp.zeros_like(counts_acc)
        loss_acc[...] = jnp.zeros_like(loss_acc)

    counts_new = counts_acc[...]
    loss_new = loss_acc[...]

    for p in range(_P):
        xt = x_ref[p]
        x2 = jnp.sum(xt * xt, axis=0, keepdims=True)
        mm = jax.lax.dot_general(cb, xt, (((1,), (0,)), ((), ())),
                                 preferred_element_type=jnp.float32)
        d = (x2 + c2) - 2.0 * mm
        min_d = jnp.min(d, axis=0, keepdims=True)
        t = jnp.where(d == min_d, rev, 0.0)
        rmax = jnp.max(t, axis=0, keepdims=True)
        oh = jnp.where(t == rmax, 1.0, 0.0)
        idx_ref[pl.ds(p * _SEQ, _SEQ)] = (
            (128.0 - rmax).astype(jnp.int32).reshape(_SEQ))
        counts_new += jax.lax.dot_general(
            oh, ones_s, (((1,), (0,)), ((), ())),
            preferred_element_type=jnp.float32)
        loss_new += jax.lax.dot_general(
            min_d, ones_s, (((1,), (0,)), ((), ())),
            preferred_element_type=jnp.float32)

    counts_acc[...] = counts_new
    loss_acc[...] = loss_new

    @pl.when(i == _NSTEPS - 1)
    def _finalize():
        loss_ref[...] = 1.25 * _INV_ND * loss_acc[...]
        avg = counts_acc[...] * (1.0 / _N_ROWS)
        ent = jnp.sum(avg * jnp.log(avg + 1e-10), axis=0, keepdims=True)
        perp_ref[...] = jnp.exp(-ent)


def _tc_call(xt, codebook):
    return pl.pallas_call(
        _vq_body,
        grid=(_NSTEPS,),
        in_specs=[
            pl.BlockSpec((_P, _DIM, _SEQ), lambda i: (i, 0, 0)),
            pl.BlockSpec((_NUM_EMB, _DIM), lambda i: (0, 0)),
        ],
        out_specs=[
            pl.BlockSpec((_P * _SEQ,), lambda i: (i,)),
            pl.BlockSpec((1, 1), lambda i: (0, 0)),
            pl.BlockSpec((1, 1), lambda i: (0, 0)),
        ],
        out_shape=[
            jax.ShapeDtypeStruct((_N_ROWS,), jnp.int32),
            jax.ShapeDtypeStruct((1, 1), jnp.float32),
            jax.ShapeDtypeStruct((1, 1), jnp.float32),
        ],
        scratch_shapes=[
            pltpu.VMEM((_NUM_EMB, 1), jnp.float32),
            pltpu.VMEM((1, 1), jnp.float32),
        ],
        compiler_params=pltpu.CompilerParams(
            dimension_semantics=("arbitrary",)),
    )(xt, codebook)


_NW = 32
_ROWS_PER_W = _N_ROWS // _NW
_CHUNK = 128
_NCHUNK = _ROWS_PER_W // _CHUNK


def _sc_gather(cb, idx):
    mesh = plsc.VectorSubcoreMesh(core_axis_name="c", subcore_axis_name="s")

    @functools.partial(
        pl.kernel,
        out_type=jax.ShapeDtypeStruct((_N_ROWS, _NUM_EMB), jnp.float32),
        mesh=mesh,
        scratch_types=[
            pltpu.VMEM((_CHUNK,), jnp.int32),
            pltpu.VMEM((_CHUNK, _NUM_EMB), jnp.float32),
            pltpu.SemaphoreType.DMA,
        ],
    )
    def k(cb_hbm, idx_hbm, out_hbm, idx_v, rows_v, sem):
        wid = lax.axis_index("s") * 2 + lax.axis_index("c")
        base = wid * _ROWS_PER_W

        def body(kk, _):
            off = base + kk * _CHUNK
            pltpu.sync_copy(idx_hbm.at[pl.ds(off, _CHUNK)], idx_v)
            pltpu.async_copy(cb_hbm.at[idx_v], rows_v, sem).wait()
            pltpu.sync_copy(rows_v, out_hbm.at[pl.ds(off, _CHUNK)])
            return ()

        lax.fori_loop(0, _NCHUNK, body, ())

    return k(cb, idx)


def kernel(inputs, codebook):
    xt = jnp.transpose(inputs, (0, 2, 1))
    idx, loss, perp = _tc_call(xt, codebook)
    cb_pad = jnp.pad(codebook, ((0, 0), (0, _NUM_EMB - _DIM)))
    q_rm = _sc_gather(cb_pad, idx)
    q = q_rm[:, :_DIM].reshape(_BATCH, _SEQ, _DIM)
    return (loss[0, 0], q, perp[0, 0], idx)


# speedup vs baseline: 4.6372x; 4.6372x over previous
import jax
import jax.numpy as jnp
from jax.experimental import pallas as pl
from jax.experimental.pallas import tpu as pltpu

_NUM_EMB = 128
_DIM = 64
_BATCH = 64
_SEQ = 1024
_P = 8
_NSTEPS = _BATCH // _P
_N_ROWS = _BATCH * _SEQ
_INV_ND = 1.0 / (_N_ROWS * _DIM)


def _vq_body(x_ref, cb_ref, idx_ref, q_ref, loss_ref, perp_ref,
             counts_acc, loss_acc):
    i = pl.program_id(0)
    cb = cb_ref[...]
    c2 = jnp.sum(cb * cb, axis=1, keepdims=True)
    rev = 128.0 - jax.lax.broadcasted_iota(
        jnp.int32, (_NUM_EMB, 1), 0).astype(jnp.float32)
    ones_s = jnp.ones((_SEQ, 1), jnp.float32)

    @pl.when(i == 0)
    def _init():
        counts_acc[...] = jnp.zeros_like(counts_acc)
        loss_acc[...] = jnp.zeros_like(loss_acc)

    counts_new = counts_acc[...]
    loss_new = loss_acc[...]

    for p in range(_P):
        xt = x_ref[p]
        x2 = jnp.sum(xt * xt, axis=0, keepdims=True)
        mm = jax.lax.dot_general(cb, xt, (((1,), (0,)), ((), ())),
                                 preferred_element_type=jnp.float32)
        d = (x2 + c2) - 2.0 * mm
        min_d = jnp.min(d, axis=0, keepdims=True)
        t = jnp.where(d == min_d, rev, 0.0)
        rmax = jnp.max(t, axis=0, keepdims=True)
        oh = jnp.where(t == rmax, 1.0, 0.0)
        qt = jax.lax.dot_general(cb, oh, (((0,), (0,)), ((), ())),
                                 preferred_element_type=jnp.float32)
        idx_ref[pl.ds(p * _SEQ, _SEQ)] = (
            (128.0 - rmax).astype(jnp.int32).reshape(_SEQ))
        q_ref[p] = qt
        counts_new += jax.lax.dot_general(
            oh, ones_s, (((1,), (0,)), ((), ())),
            preferred_element_type=jnp.float32)
        loss_new += jax.lax.dot_general(
            min_d, ones_s, (((1,), (0,)), ((), ())),
            preferred_element_type=jnp.float32)

    counts_acc[...] = counts_new
    loss_acc[...] = loss_new

    @pl.when(i == _NSTEPS - 1)
    def _finalize():
        loss_ref[...] = 1.25 * _INV_ND * loss_acc[...]
        avg = counts_acc[...] * (1.0 / _N_ROWS)
        ent = jnp.sum(avg * jnp.log(avg + 1e-10), axis=0, keepdims=True)
        perp_ref[...] = jnp.exp(-ent)


def kernel(inputs, codebook):
    xt = jnp.transpose(inputs, (0, 2, 1))
    idx, qt, loss, perp = pl.pallas_call(
        _vq_body,
        grid=(_NSTEPS,),
        in_specs=[
            pl.BlockSpec((_P, _DIM, _SEQ), lambda i: (i, 0, 0)),
            pl.BlockSpec((_NUM_EMB, _DIM), lambda i: (0, 0)),
        ],
        out_specs=[
            pl.BlockSpec((_P * _SEQ,), lambda i: (i,)),
            pl.BlockSpec((_P, _DIM, _SEQ), lambda i: (i, 0, 0)),
            pl.BlockSpec((1, 1), lambda i: (0, 0)),
            pl.BlockSpec((1, 1), lambda i: (0, 0)),
        ],
        out_shape=[
            jax.ShapeDtypeStruct((_N_ROWS,), jnp.int32),
            jax.ShapeDtypeStruct((_BATCH, _DIM, _SEQ), jnp.float32),
            jax.ShapeDtypeStruct((1, 1), jnp.float32),
            jax.ShapeDtypeStruct((1, 1), jnp.float32),
        ],
        scratch_shapes=[
            pltpu.VMEM((_NUM_EMB, 1), jnp.float32),
            pltpu.VMEM((1, 1), jnp.float32),
        ],
        compiler_params=pltpu.CompilerParams(
            dimension_semantics=("arbitrary",)),
    )(xt, codebook)
    q = jnp.transpose(qt, (0, 2, 1))
    return (loss[0, 0], q, perp[0, 0], idx)
